# baseline (speedup 1.0000x reference)
import numpy as np
import jax
import jax.numpy as jnp
from jax.experimental import pallas as pl
from jax.experimental.pallas import tpu as pltpu

D = 768
H = 12
DH = 64
E = 8
V = 10000
T = 2048
F = 4 * D


def _ln(x, g, b, eps=1e-5):
    m = jnp.mean(x, axis=-1, keepdims=True)
    v = jnp.mean((x - m) ** 2, axis=-1, keepdims=True)
    return (x - m) * jax.lax.rsqrt(v + eps) * g + b


def _mxu(a, b, dn):
    return jax.lax.dot_general(a, b, dn, preferred_element_type=jnp.float32)


def _mm_t(a, w):
    return _mxu(a, w, (((1,), (1,)), ((), ())))


def _mm(a, w):
    return _mxu(a, w, (((1,), (0,)), ((), ())))


def _ln_mm_kern(x_ref, g_ref, b_ref, w_ref, wb_ref, o_ref):
    y = _ln(x_ref[...], g_ref[...], b_ref[...])
    o_ref[...] = _mm_t(y, w_ref[...]) + wb_ref[...]


def _ln_mm(x, g, b, w, wb, bt=256):
    n = w.shape[0]
    return pl.pallas_call(
        _ln_mm_kern,
        grid=(T // bt,),
        in_specs=[
            pl.BlockSpec((bt, D), lambda t: (t, 0)),
            pl.BlockSpec((1, D), lambda t: (0, 0)),
            pl.BlockSpec((1, D), lambda t: (0, 0)),
            pl.BlockSpec((n, D), lambda t: (0, 0)),
            pl.BlockSpec((1, n), lambda t: (0, 0)),
        ],
        out_specs=pl.BlockSpec((bt, n), lambda t: (t, 0)),
        out_shape=jax.ShapeDtypeStruct((T, n), jnp.float32),
    )(x, g.reshape(1, D), b.reshape(1, D), w, wb.reshape(1, n))


def _attn_kern(his_ref, q_ref, k_ref, v_ref, o_ref):
    t = pl.program_id(1)
    bt = q_ref.shape[1]
    s = _mm_t(q_ref[0], k_ref[0]) / np.sqrt(DH)
    row = t * bt + jax.lax.broadcasted_iota(jnp.int32, s.shape, 0)
    col = jax.lax.broadcasted_iota(jnp.int32, s.shape, 1)
    pad = his_ref[...] == 0
    s = jnp.where((col > row) | pad, -1e30, s)
    m = jnp.max(s, axis=-1, keepdims=True)
    e = jnp.exp(s - m)
    den = jnp.sum(e, axis=-1, keepdims=True)
    o_ref[0] = _mm(e, v_ref[0]) / den


def _attention(qkvh, his, bt=256):
    return pl.pallas_call(
        _attn_kern,
        grid=(H, T // bt),
        in_specs=[
            pl.BlockSpec((1, T), lambda h, t: (0, 0)),
            pl.BlockSpec((1, bt, DH), lambda h, t: (h, t, 0)),
            pl.BlockSpec((1, T, DH), lambda h, t: (H + h, 0, 0)),
            pl.BlockSpec((1, T, DH), lambda h, t: (2 * H + h, 0, 0)),
        ],
        out_specs=pl.BlockSpec((1, bt, DH), lambda h, t: (h, t, 0)),
        out_shape=jax.ShapeDtypeStruct((H, T, DH), jnp.float32),
    )(his, qkvh, qkvh, qkvh)


def _proj_res_kern(a_ref, w_ref, b_ref, x_ref, o_ref):
    o_ref[...] = x_ref[...] + _mm_t(a_ref[...], w_ref[...]) + b_ref[...]


def _proj_res(a, w, wb, x, bt=256):
    return pl.pallas_call(
        _proj_res_kern,
        grid=(T // bt,),
        in_specs=[
            pl.BlockSpec((bt, D), lambda t: (t, 0)),
            pl.BlockSpec((D, D), lambda t: (0, 0)),
            pl.BlockSpec((1, D), lambda t: (0, 0)),
            pl.BlockSpec((bt, D), lambda t: (t, 0)),
        ],
        out_specs=pl.BlockSpec((bt, D), lambda t: (t, 0)),
        out_shape=jax.ShapeDtypeStruct((T, D), jnp.float32),
    )(a, w, wb.reshape(1, D), x)


def _router_kern(x_ref, g_ref, b_ref, rw_ref, rb_ref, nw_ref, nb_ref,
                 nz_ref, y_ref, gate_ref):
    y = _ln(x_ref[...], g_ref[...], b_ref[...])
    y_ref[...] = y
    lg = _mm_t(y, rw_ref[...]) + rb_ref[...]
    nl = _mm_t(y, nw_ref[...]) + nb_ref[...]
    noisy = lg + nz_ref[...] * jax.nn.softplus(nl)
    m1 = jnp.max(noisy, axis=-1, keepdims=True)
    top1 = noisy == m1
    m2 = jnp.max(jnp.where(top1, -jnp.inf, noisy), axis=-1, keepdims=True)
    sel = noisy >= m2
    gate_ref[...] = jax.nn.softmax(jnp.where(sel, noisy, -jnp.inf), axis=-1)


def _router(x, g, b, rw, rb, nw, nb, nz, bt=256):
    return pl.pallas_call(
        _router_kern,
        grid=(T // bt,),
        in_specs=[
            pl.BlockSpec((bt, D), lambda t: (t, 0)),
            pl.BlockSpec((1, D), lambda t: (0, 0)),
            pl.BlockSpec((1, D), lambda t: (0, 0)),
            pl.BlockSpec((E, D), lambda t: (0, 0)),
            pl.BlockSpec((1, E), lambda t: (0, 0)),
            pl.BlockSpec((E, D), lambda t: (0, 0)),
            pl.BlockSpec((1, E), lambda t: (0, 0)),
            pl.BlockSpec((bt, E), lambda t: (t, 0)),
        ],
        out_specs=[
            pl.BlockSpec((bt, D), lambda t: (t, 0)),
            pl.BlockSpec((bt, E), lambda t: (t, 0)),
        ],
        out_shape=[
            jax.ShapeDtypeStruct((T, D), jnp.float32),
            jax.ShapeDtypeStruct((T, E), jnp.float32),
        ],
    )(x, g.reshape(1, D), b.reshape(1, D), rw, rb.reshape(1, E),
      nw, nb.reshape(1, E), nz)


def _moe_kern(y_ref, gate_ref, w1_ref, b1_ref, w2_ref, b2_ref, x_ref,
              o_ref, acc_ref):
    e = pl.program_id(0)
    f = pl.program_id(1)
    t = pl.program_id(2)
    nf = pl.num_programs(1)
    bt = y_ref.shape[0]
    sl = pl.ds(t * bt, bt)

    @pl.when((e == 0) & (f == 0))
    def _():
        acc_ref[sl, :] = x_ref[...]

    h = _mm(y_ref[...], w1_ref[0]) + b1_ref[0]
    h = jax.nn.gelu(h, approximate=True)
    o = _mm(h, w2_ref[0])
    o = o + jnp.where(f == 0, 1.0, 0.0) * b2_ref[0]
    col = jax.lax.broadcasted_iota(jnp.int32, (bt, E), 1)
    g = jnp.sum(jnp.where(col == e, gate_ref[...], 0.0), axis=1, keepdims=True)
    acc_ref[sl, :] += o * g

    @pl.when((e == E - 1) & (f == nf - 1))
    def _():
        o_ref[...] = acc_ref[sl, :]


def _moe(y, gate, w1, b1, w2, b2, x, bt=256, nf=2):
    bf = F // nf
    return pl.pallas_call(
        _moe_kern,
        grid=(E, nf, T // bt),
        in_specs=[
            pl.BlockSpec((bt, D), lambda e, f, t: (t, 0)),
            pl.BlockSpec((bt, E), lambda e, f, t: (t, 0)),
            pl.BlockSpec((1, D, bf), lambda e, f, t: (e, 0, f)),
            pl.BlockSpec((1, 1, bf), lambda e, f, t: (e, 0, f)),
            pl.BlockSpec((1, bf, D), lambda e, f, t: (e, f, 0)),
            pl.BlockSpec((1, 1, D), lambda e, f, t: (e, 0, 0)),
            pl.BlockSpec((bt, D), lambda e, f, t: (t, 0)),
        ],
        out_specs=pl.BlockSpec((bt, D), lambda e, f, t: (t, 0)),
        out_shape=jax.ShapeDtypeStruct((T, D), jnp.float32),
        scratch_shapes=[pltpu.VMEM((T, D), jnp.float32)],
    )(y, gate, w1, b1.reshape(E, 1, F), w2, b2.reshape(E, 1, D), x)


def _tower_kern(v_ref, w_ref, b_ref, o_ref):
    o_ref[...] = _mm_t(v_ref[...], w_ref[...]) + b_ref[...]


def _tower(vocab, w, wb, bv=1000):
    return pl.pallas_call(
        _tower_kern,
        grid=(V // bv,),
        in_specs=[
            pl.BlockSpec((bv, 31), lambda i: (i, 0)),
            pl.BlockSpec((D, 31), lambda i: (0, 0)),
            pl.BlockSpec((1, D), lambda i: (0, 0)),
        ],
        out_specs=pl.BlockSpec((bv, D), lambda i: (i, 0)),
        out_shape=jax.ShapeDtypeStruct((V, D), jnp.float32),
    )(vocab, w, wb.reshape(1, D))


def _head_kern(xf_ref, le_ref, tgt_ref, lg_ref, num_ref, den_ref):
    t = pl.program_id(0)
    logits = _mm_t(xf_ref[...], le_ref[...])
    lg_ref[...] = logits
    mx = jnp.max(logits, axis=-1, keepdims=True)
    lse = mx + jnp.log(jnp.sum(jnp.exp(logits - mx), axis=-1, keepdims=True))
    tgt = tgt_ref[...]
    colv = jax.lax.broadcasted_iota(jnp.int32, logits.shape, 1)
    tl = jnp.sum(jnp.where(colv == tgt, logits, 0.0), axis=1, keepdims=True)
    nll = lse - tl
    m = (tgt != 0).astype(jnp.float32)

    @pl.when(t == 0)
    def _():
        num_ref[...] = jnp.zeros_like(num_ref)
        den_ref[...] = jnp.zeros_like(den_ref)

    num_ref[...] += jnp.sum(nll * m).reshape(1, 1)
    den_ref[...] += jnp.sum(m).reshape(1, 1)


def _head(xf, le, tgt, bt=128):
    return pl.pallas_call(
        _head_kern,
        grid=(T // bt,),
        in_specs=[
            pl.BlockSpec((bt, D), lambda t: (t, 0)),
            pl.BlockSpec((V, D), lambda t: (0, 0)),
            pl.BlockSpec((bt, 1), lambda t: (t, 0)),
        ],
        out_specs=[
            pl.BlockSpec((bt, V), lambda t: (t, 0)),
            pl.BlockSpec((1, 1), lambda t: (0, 0)),
            pl.BlockSpec((1, 1), lambda t: (0, 0)),
        ],
        out_shape=[
            jax.ShapeDtypeStruct((T, V), jnp.float32),
            jax.ShapeDtypeStruct((1, 1), jnp.float32),
            jax.ShapeDtypeStruct((1, 1), jnp.float32),
        ],
    )(xf, le, tgt)


def _embed(params, his, ts, vocab):
    loc = vocab[his]
    poi = loc[..., :28]
    lonlat = loc[..., 28:30]
    rank = loc[..., 30].astype(jnp.int32)
    poi_e = poi @ params['poi_W'].T + params['poi_b']
    ll_e = lonlat @ params['lonlat_W'].T + params['lonlat_b']
    r_e = params['rank_emb'][rank]
    tok = jnp.concatenate([ll_e, r_e, poi_e], axis=-1)
    x = tok + params['time_emb'][ts] + params['wpe'][None, :ts.shape[1], :]
    return x


def kernel(params, his, ts, targets, vocab):
    p = params
    x = _embed(p, his, ts, vocab)[0]
    for li, bp in enumerate(p['blocks']):
        qkv = _ln_mm(x, bp['ln1_g'], bp['ln1_b'], bp['in_proj_W'],
                     bp['in_proj_b'])
        qkvh = qkv.reshape(T, 3 * H, DH).transpose(1, 0, 2)
        a = _attention(qkvh, his)
        a = a.transpose(1, 0, 2).reshape(T, D)
        x = _proj_res(a, bp['out_W'], bp['out_b'], x)
        nz = jax.random.normal(jax.random.fold_in(jax.random.key(42), li),
                               (1, T, E), jnp.float32)[0]
        y, gate = _router(x, bp['ln2_g'], bp['ln2_b'], bp['route_W'],
                          bp['route_b'], bp['noise_W'], bp['noise_b'], nz)
        x = _moe(y, gate, bp['W1'], bp['b1'], bp['W2'], bp['b2'], x)
    xf = _ln_mm(x, p['lnf_g'], p['lnf_b'], p['lm_head_W'],
                jnp.zeros((D,), jnp.float32))
    le = _tower(vocab, p['tower_W'], p['tower_b'])
    logits, num, den = _head(xf, le, targets.reshape(T, 1))
    loss = (num / jnp.maximum(den, 1.0)).reshape(())
    return logits.reshape(1, T, V), loss


# speedup vs baseline: 1.0997x; 1.0997x over previous
import numpy as np
import jax
import jax.numpy as jnp
from jax.experimental import pallas as pl
from jax.experimental.pallas import tpu as pltpu

D = 768
H = 12
DH = 64
E = 8
V = 10000
T = 2048
F = 4 * D


def _ln(x, g, b, eps=1e-5):
    m = jnp.mean(x, axis=-1, keepdims=True)
    v = jnp.mean((x - m) ** 2, axis=-1, keepdims=True)
    return (x - m) * jax.lax.rsqrt(v + eps) * g + b


def _mxu(a, b, dn):
    return jax.lax.dot_general(a, b, dn, preferred_element_type=jnp.float32)


def _mm_t(a, w):
    return _mxu(a, w, (((1,), (1,)), ((), ())))


def _mm(a, w):
    return _mxu(a, w, (((1,), (0,)), ((), ())))


def _ln_mm_kern(x_ref, g_ref, b_ref, w_ref, wb_ref, o_ref):
    y = _ln(x_ref[...], g_ref[...], b_ref[...])
    o_ref[...] = _mm_t(y, w_ref[...]) + wb_ref[...]


def _ln_mm(x, g, b, w, wb, bt=512):
    n = w.shape[0]
    return pl.pallas_call(
        _ln_mm_kern,
        grid=(T // bt,),
        in_specs=[
            pl.BlockSpec((bt, D), lambda t: (t, 0)),
            pl.BlockSpec((1, D), lambda t: (0, 0)),
            pl.BlockSpec((1, D), lambda t: (0, 0)),
            pl.BlockSpec((n, D), lambda t: (0, 0)),
            pl.BlockSpec((1, n), lambda t: (0, 0)),
        ],
        out_specs=pl.BlockSpec((bt, n), lambda t: (t, 0)),
        out_shape=jax.ShapeDtypeStruct((T, n), jnp.float32),
    )(x, g.reshape(1, D), b.reshape(1, D), w, wb.reshape(1, n))


def _attn_kern(his_ref, q_ref, k_ref, v_ref, o_ref):
    t = pl.program_id(1)
    bt = q_ref.shape[1]
    s = _mm_t(q_ref[0], k_ref[0]) / np.sqrt(DH)
    row = t * bt + jax.lax.broadcasted_iota(jnp.int32, s.shape, 0)
    col = jax.lax.broadcasted_iota(jnp.int32, s.shape, 1)
    pad = his_ref[...] == 0
    s = jnp.where((col > row) | pad, -1e30, s)
    m = jnp.max(s, axis=-1, keepdims=True)
    e = jnp.exp(s - m)
    den = jnp.sum(e, axis=-1, keepdims=True)
    o_ref[0] = _mm(e, v_ref[0]) / den


def _attention(qkvh, his, bt=512):
    return pl.pallas_call(
        _attn_kern,
        grid=(H, T // bt),
        in_specs=[
            pl.BlockSpec((1, T), lambda h, t: (0, 0)),
            pl.BlockSpec((1, bt, DH), lambda h, t: (h, t, 0)),
            pl.BlockSpec((1, T, DH), lambda h, t: (H + h, 0, 0)),
            pl.BlockSpec((1, T, DH), lambda h, t: (2 * H + h, 0, 0)),
        ],
        out_specs=pl.BlockSpec((1, bt, DH), lambda h, t: (h, t, 0)),
        out_shape=jax.ShapeDtypeStruct((H, T, DH), jnp.float32),
    )(his, qkvh, qkvh, qkvh)


def _proj_res_kern(a_ref, w_ref, b_ref, x_ref, o_ref):
    o_ref[...] = x_ref[...] + _mm_t(a_ref[...], w_ref[...]) + b_ref[...]


def _proj_res(a, w, wb, x, bt=512):
    return pl.pallas_call(
        _proj_res_kern,
        grid=(T // bt,),
        in_specs=[
            pl.BlockSpec((bt, D), lambda t: (t, 0)),
            pl.BlockSpec((D, D), lambda t: (0, 0)),
            pl.BlockSpec((1, D), lambda t: (0, 0)),
            pl.BlockSpec((bt, D), lambda t: (t, 0)),
        ],
        out_specs=pl.BlockSpec((bt, D), lambda t: (t, 0)),
        out_shape=jax.ShapeDtypeStruct((T, D), jnp.float32),
    )(a, w, wb.reshape(1, D), x)


def _router_kern(x_ref, g_ref, b_ref, rw_ref, rb_ref, nw_ref, nb_ref,
                 nz_ref, y_ref, gate_ref):
    y = _ln(x_ref[...], g_ref[...], b_ref[...])
    y_ref[...] = y
    lg = _mm_t(y, rw_ref[...]) + rb_ref[...]
    nl = _mm_t(y, nw_ref[...]) + nb_ref[...]
    noisy = lg + nz_ref[...] * jax.nn.softplus(nl)
    m1 = jnp.max(noisy, axis=-1, keepdims=True)
    top1 = noisy == m1
    m2 = jnp.max(jnp.where(top1, -jnp.inf, noisy), axis=-1, keepdims=True)
    sel = noisy >= m2
    gate_ref[...] = jax.nn.softmax(jnp.where(sel, noisy, -jnp.inf), axis=-1)


def _router(x, g, b, rw, rb, nw, nb, nz, bt=512):
    return pl.pallas_call(
        _router_kern,
        grid=(T // bt,),
        in_specs=[
            pl.BlockSpec((bt, D), lambda t: (t, 0)),
            pl.BlockSpec((1, D), lambda t: (0, 0)),
            pl.BlockSpec((1, D), lambda t: (0, 0)),
            pl.BlockSpec((E, D), lambda t: (0, 0)),
            pl.BlockSpec((1, E), lambda t: (0, 0)),
            pl.BlockSpec((E, D), lambda t: (0, 0)),
            pl.BlockSpec((1, E), lambda t: (0, 0)),
            pl.BlockSpec((bt, E), lambda t: (t, 0)),
        ],
        out_specs=[
            pl.BlockSpec((bt, D), lambda t: (t, 0)),
            pl.BlockSpec((bt, E), lambda t: (t, 0)),
        ],
        out_shape=[
            jax.ShapeDtypeStruct((T, D), jnp.float32),
            jax.ShapeDtypeStruct((T, E), jnp.float32),
        ],
    )(x, g.reshape(1, D), b.reshape(1, D), rw, rb.reshape(1, E),
      nw, nb.reshape(1, E), nz)


def _moe_kern(y_ref, gate_ref, w1_ref, b1_ref, w2_ref, b2_ref, x_ref,
              o_ref, acc_ref):
    e = pl.program_id(0)
    f = pl.program_id(1)
    t = pl.program_id(2)
    nf = pl.num_programs(1)
    bt = y_ref.shape[0]
    sl = pl.ds(t * bt, bt)

    @pl.when((e == 0) & (f == 0))
    def _():
        acc_ref[sl, :] = x_ref[...]

    h = _mm(y_ref[...], w1_ref[0]) + b1_ref[0]
    h = jax.nn.gelu(h, approximate=True)
    o = _mm(h, w2_ref[0])
    o = o + jnp.where(f == 0, 1.0, 0.0) * b2_ref[0]
    col = jax.lax.broadcasted_iota(jnp.int32, (bt, E), 1)
    g = jnp.sum(jnp.where(col == e, gate_ref[...], 0.0), axis=1, keepdims=True)
    acc_ref[sl, :] += o * g

    @pl.when((e == E - 1) & (f == nf - 1))
    def _():
        o_ref[...] = acc_ref[sl, :]


def _moe(y, gate, w1, b1, w2, b2, x, bt=512, nf=2):
    bf = F // nf
    return pl.pallas_call(
        _moe_kern,
        grid=(E, nf, T // bt),
        in_specs=[
            pl.BlockSpec((bt, D), lambda e, f, t: (t, 0)),
            pl.BlockSpec((bt, E), lambda e, f, t: (t, 0)),
            pl.BlockSpec((1, D, bf), lambda e, f, t: (e, 0, f)),
            pl.BlockSpec((1, 1, bf), lambda e, f, t: (e, 0, f)),
            pl.BlockSpec((1, bf, D), lambda e, f, t: (e, f, 0)),
            pl.BlockSpec((1, 1, D), lambda e, f, t: (e, 0, 0)),
            pl.BlockSpec((bt, D), lambda e, f, t: (t, 0)),
        ],
        out_specs=pl.BlockSpec((bt, D), lambda e, f, t: (t, 0)),
        out_shape=jax.ShapeDtypeStruct((T, D), jnp.float32),
        scratch_shapes=[pltpu.VMEM((T, D), jnp.float32)],
    )(y, gate, w1, b1.reshape(E, 1, F), w2, b2.reshape(E, 1, D), x)


def _tower_kern(v_ref, w_ref, b_ref, o_ref):
    o_ref[...] = _mm_t(v_ref[...], w_ref[...]) + b_ref[...]


def _tower(vocab, w, wb, bv=1000):
    return pl.pallas_call(
        _tower_kern,
        grid=(V // bv,),
        in_specs=[
            pl.BlockSpec((bv, 31), lambda i: (i, 0)),
            pl.BlockSpec((D, 31), lambda i: (0, 0)),
            pl.BlockSpec((1, D), lambda i: (0, 0)),
        ],
        out_specs=pl.BlockSpec((bv, D), lambda i: (i, 0)),
        out_shape=jax.ShapeDtypeStruct((V, D), jnp.float32),
    )(vocab, w, wb.reshape(1, D))


def _head_kern(xf_ref, le_ref, tgt_ref, lg_ref, num_ref, den_ref):
    t = pl.program_id(0)
    logits = _mm_t(xf_ref[...], le_ref[...])
    lg_ref[...] = logits
    mx = jnp.max(logits, axis=-1, keepdims=True)
    lse = mx + jnp.log(jnp.sum(jnp.exp(logits - mx), axis=-1, keepdims=True))
    tgt = tgt_ref[...]
    colv = jax.lax.broadcasted_iota(jnp.int32, logits.shape, 1)
    tl = jnp.sum(jnp.where(colv == tgt, logits, 0.0), axis=1, keepdims=True)
    nll = lse - tl
    m = (tgt != 0).astype(jnp.float32)

    @pl.when(t == 0)
    def _():
        num_ref[...] = jnp.zeros_like(num_ref)
        den_ref[...] = jnp.zeros_like(den_ref)

    num_ref[...] += jnp.sum(nll * m).reshape(1, 1)
    den_ref[...] += jnp.sum(m).reshape(1, 1)


def _head(xf, le, tgt, bt=128):
    return pl.pallas_call(
        _head_kern,
        grid=(T // bt,),
        in_specs=[
            pl.BlockSpec((bt, D), lambda t: (t, 0)),
            pl.BlockSpec((V, D), lambda t: (0, 0)),
            pl.BlockSpec((bt, 1), lambda t: (t, 0)),
        ],
        out_specs=[
            pl.BlockSpec((bt, V), lambda t: (t, 0)),
            pl.BlockSpec((1, 1), lambda t: (0, 0)),
            pl.BlockSpec((1, 1), lambda t: (0, 0)),
        ],
        out_shape=[
            jax.ShapeDtypeStruct((T, V), jnp.float32),
            jax.ShapeDtypeStruct((1, 1), jnp.float32),
            jax.ShapeDtypeStruct((1, 1), jnp.float32),
        ],
    )(xf, le, tgt)


def _embed(params, his, ts, vocab):
    loc = vocab[his]
    poi = loc[..., :28]
    lonlat = loc[..., 28:30]
    rank = loc[..., 30].astype(jnp.int32)
    poi_e = poi @ params['poi_W'].T + params['poi_b']
    ll_e = lonlat @ params['lonlat_W'].T + params['lonlat_b']
    r_e = params['rank_emb'][rank]
    tok = jnp.concatenate([ll_e, r_e, poi_e], axis=-1)
    x = tok + params['time_emb'][ts] + params['wpe'][None, :ts.shape[1], :]
    return x


def kernel(params, his, ts, targets, vocab):
    p = params
    x = _embed(p, his, ts, vocab)[0]
    for li, bp in enumerate(p['blocks']):
        qkv = _ln_mm(x, bp['ln1_g'], bp['ln1_b'], bp['in_proj_W'],
                     bp['in_proj_b'])
        qkvh = qkv.reshape(T, 3 * H, DH).transpose(1, 0, 2)
        a = _attention(qkvh, his)
        a = a.transpose(1, 0, 2).reshape(T, D)
        x = _proj_res(a, bp['out_W'], bp['out_b'], x)
        nz = jax.random.normal(jax.random.fold_in(jax.random.key(42), li),
                               (1, T, E), jnp.float32)[0]
        y, gate = _router(x, bp['ln2_g'], bp['ln2_b'], bp['route_W'],
                          bp['route_b'], bp['noise_W'], bp['noise_b'], nz)
        x = _moe(y, gate, bp['W1'], bp['b1'], bp['W2'], bp['b2'], x)
    xf = _ln_mm(x, p['lnf_g'], p['lnf_b'], p['lm_head_W'],
                jnp.zeros((D,), jnp.float32))
    le = _tower(vocab, p['tower_W'], p['tower_b'])
    logits, num, den = _head(xf, le, targets.reshape(T, 1))
    loss = (num / jnp.maximum(den, 1.0)).reshape(())
    return logits.reshape(1, T, V), loss


# speedup vs baseline: 1.1580x; 1.0530x over previous
import numpy as np
import jax
import jax.numpy as jnp
from jax.experimental import pallas as pl
from jax.experimental.pallas import tpu as pltpu

D = 768
H = 12
DH = 64
E = 8
V = 10000
T = 2048
F = 4 * D


def _ln(x, g, b, eps=1e-5):
    m = jnp.mean(x, axis=-1, keepdims=True)
    v = jnp.mean((x - m) ** 2, axis=-1, keepdims=True)
    return (x - m) * jax.lax.rsqrt(v + eps) * g + b


def _mxu(a, b, dn):
    return jax.lax.dot_general(a, b, dn, preferred_element_type=jnp.float32)


def _mm_t(a, w):
    return _mxu(a, w, (((1,), (1,)), ((), ())))


def _mm(a, w):
    return _mxu(a, w, (((1,), (0,)), ((), ())))


def _ln_mm_kern(x_ref, g_ref, b_ref, w_ref, wb_ref, o_ref):
    y = _ln(x_ref[...], g_ref[...], b_ref[...])
    o_ref[...] = _mm_t(y, w_ref[...]) + wb_ref[...]


def _ln_mm(x, g, b, w, wb, bt=512):
    n = w.shape[0]
    return pl.pallas_call(
        _ln_mm_kern,
        grid=(T // bt,),
        in_specs=[
            pl.BlockSpec((bt, D), lambda t: (t, 0)),
            pl.BlockSpec((1, D), lambda t: (0, 0)),
            pl.BlockSpec((1, D), lambda t: (0, 0)),
            pl.BlockSpec((n, D), lambda t: (0, 0)),
            pl.BlockSpec((1, n), lambda t: (0, 0)),
        ],
        out_specs=pl.BlockSpec((bt, n), lambda t: (t, 0)),
        out_shape=jax.ShapeDtypeStruct((T, n), jnp.float32),
    )(x, g.reshape(1, D), b.reshape(1, D), w, wb.reshape(1, n))


def _attn_kern(his_ref, q_ref, k_ref, v_ref, o_ref):
    t = pl.program_id(1)
    bt = q_ref.shape[1]
    s = _mm_t(q_ref[0], k_ref[0]) / np.sqrt(DH)
    row = t * bt + jax.lax.broadcasted_iota(jnp.int32, s.shape, 0)
    col = jax.lax.broadcasted_iota(jnp.int32, s.shape, 1)
    pad = his_ref[...] == 0
    s = jnp.where((col > row) | pad, -1e30, s)
    m = jnp.max(s, axis=-1, keepdims=True)
    e = jnp.exp(s - m)
    den = jnp.sum(e, axis=-1, keepdims=True)
    o_ref[0] = _mm(e, v_ref[0]) / den


def _attention(qkvh, his, bt=512):
    return pl.pallas_call(
        _attn_kern,
        grid=(H, T // bt),
        in_specs=[
            pl.BlockSpec((1, T), lambda h, t: (0, 0)),
            pl.BlockSpec((1, bt, DH), lambda h, t: (h, t, 0)),
            pl.BlockSpec((1, T, DH), lambda h, t: (H + h, 0, 0)),
            pl.BlockSpec((1, T, DH), lambda h, t: (2 * H + h, 0, 0)),
        ],
        out_specs=pl.BlockSpec((1, bt, DH), lambda h, t: (h, t, 0)),
        out_shape=jax.ShapeDtypeStruct((H, T, DH), jnp.float32),
    )(his, qkvh, qkvh, qkvh)


def _proj_res_kern(a_ref, w_ref, b_ref, x_ref, o_ref):
    o_ref[...] = x_ref[...] + _mm_t(a_ref[...], w_ref[...]) + b_ref[...]


def _proj_res(a, w, wb, x, bt=512):
    return pl.pallas_call(
        _proj_res_kern,
        grid=(T // bt,),
        in_specs=[
            pl.BlockSpec((bt, D), lambda t: (t, 0)),
            pl.BlockSpec((D, D), lambda t: (0, 0)),
            pl.BlockSpec((1, D), lambda t: (0, 0)),
            pl.BlockSpec((bt, D), lambda t: (t, 0)),
        ],
        out_specs=pl.BlockSpec((bt, D), lambda t: (t, 0)),
        out_shape=jax.ShapeDtypeStruct((T, D), jnp.float32),
    )(a, w, wb.reshape(1, D), x)


def _router_kern(x_ref, g_ref, b_ref, rw_ref, rb_ref, nw_ref, nb_ref,
                 nz_ref, y_ref, gate_ref):
    y = _ln(x_ref[...], g_ref[...], b_ref[...])
    y_ref[...] = y
    lg = _mm_t(y, rw_ref[...]) + rb_ref[...]
    nl = _mm_t(y, nw_ref[...]) + nb_ref[...]
    noisy = lg + nz_ref[...] * jax.nn.softplus(nl)
    m1 = jnp.max(noisy, axis=-1, keepdims=True)
    top1 = noisy == m1
    m2 = jnp.max(jnp.where(top1, -jnp.inf, noisy), axis=-1, keepdims=True)
    sel = noisy >= m2
    gate_ref[...] = jax.nn.softmax(jnp.where(sel, noisy, -jnp.inf), axis=-1)


def _router(x, g, b, rw, rb, nw, nb, nz, bt=512):
    return pl.pallas_call(
        _router_kern,
        grid=(T // bt,),
        in_specs=[
            pl.BlockSpec((bt, D), lambda t: (t, 0)),
            pl.BlockSpec((1, D), lambda t: (0, 0)),
            pl.BlockSpec((1, D), lambda t: (0, 0)),
            pl.BlockSpec((E, D), lambda t: (0, 0)),
            pl.BlockSpec((1, E), lambda t: (0, 0)),
            pl.BlockSpec((E, D), lambda t: (0, 0)),
            pl.BlockSpec((1, E), lambda t: (0, 0)),
            pl.BlockSpec((bt, E), lambda t: (t, 0)),
        ],
        out_specs=[
            pl.BlockSpec((bt, D), lambda t: (t, 0)),
            pl.BlockSpec((bt, E), lambda t: (t, 0)),
        ],
        out_shape=[
            jax.ShapeDtypeStruct((T, D), jnp.float32),
            jax.ShapeDtypeStruct((T, E), jnp.float32),
        ],
    )(x, g.reshape(1, D), b.reshape(1, D), rw, rb.reshape(1, E),
      nw, nb.reshape(1, E), nz)


def _moe_kern(y_ref, gate_ref, w1_ref, b1_ref, w2_ref, b2_ref, x_ref,
              o_ref, acc_ref):
    e = pl.program_id(0)
    f = pl.program_id(1)
    t = pl.program_id(2)
    nf = pl.num_programs(1)
    bt = y_ref.shape[0]
    sl = pl.ds(t * bt, bt)

    @pl.when((e == 0) & (f == 0))
    def _():
        acc_ref[sl, :] = x_ref[...]

    h = _mm(y_ref[...], w1_ref[0]) + b1_ref[0]
    h = jax.nn.gelu(h, approximate=True)
    o = _mm(h, w2_ref[0])
    o = o + jnp.where(f == 0, 1.0, 0.0) * b2_ref[0]
    col = jax.lax.broadcasted_iota(jnp.int32, (bt, E), 1)
    g = jnp.sum(jnp.where(col == e, gate_ref[...], 0.0), axis=1, keepdims=True)
    acc_ref[sl, :] += o * g

    @pl.when((e == E - 1) & (f == nf - 1))
    def _():
        o_ref[...] = acc_ref[sl, :]


def _moe(y, gate, w1, b1, w2, b2, x, bt=512, nf=1):
    bf = F // nf
    return pl.pallas_call(
        _moe_kern,
        grid=(E, nf, T // bt),
        in_specs=[
            pl.BlockSpec((bt, D), lambda e, f, t: (t, 0)),
            pl.BlockSpec((bt, E), lambda e, f, t: (t, 0)),
            pl.BlockSpec((1, D, bf), lambda e, f, t: (e, 0, f)),
            pl.BlockSpec((1, 1, bf), lambda e, f, t: (e, 0, f)),
            pl.BlockSpec((1, bf, D), lambda e, f, t: (e, f, 0)),
            pl.BlockSpec((1, 1, D), lambda e, f, t: (e, 0, 0)),
            pl.BlockSpec((bt, D), lambda e, f, t: (t, 0)),
        ],
        out_specs=pl.BlockSpec((bt, D), lambda e, f, t: (t, 0)),
        out_shape=jax.ShapeDtypeStruct((T, D), jnp.float32),
        scratch_shapes=[pltpu.VMEM((T, D), jnp.float32)],
    )(y, gate, w1, b1.reshape(E, 1, F), w2, b2.reshape(E, 1, D), x)


def _tower_kern(v_ref, w_ref, b_ref, o_ref):
    o_ref[...] = _mm_t(v_ref[...], w_ref[...]) + b_ref[...]


def _tower(vocab, w, wb, bv=1000):
    return pl.pallas_call(
        _tower_kern,
        grid=(V // bv,),
        in_specs=[
            pl.BlockSpec((bv, 31), lambda i: (i, 0)),
            pl.BlockSpec((D, 31), lambda i: (0, 0)),
            pl.BlockSpec((1, D), lambda i: (0, 0)),
        ],
        out_specs=pl.BlockSpec((bv, D), lambda i: (i, 0)),
        out_shape=jax.ShapeDtypeStruct((V, D), jnp.float32),
    )(vocab, w, wb.reshape(1, D))


def _head_kern(xf_ref, le_ref, tgt_ref, lg_ref, num_ref, den_ref):
    t = pl.program_id(0)
    logits = _mm_t(xf_ref[...], le_ref[...])
    lg_ref[...] = logits
    mx = jnp.max(logits, axis=-1, keepdims=True)
    lse = mx + jnp.log(jnp.sum(jnp.exp(logits - mx), axis=-1, keepdims=True))
    tgt = tgt_ref[...]
    colv = jax.lax.broadcasted_iota(jnp.int32, logits.shape, 1)
    tl = jnp.sum(jnp.where(colv == tgt, logits, 0.0), axis=1, keepdims=True)
    nll = lse - tl
    m = (tgt != 0).astype(jnp.float32)

    @pl.when(t == 0)
    def _():
        num_ref[...] = jnp.zeros_like(num_ref)
        den_ref[...] = jnp.zeros_like(den_ref)

    num_ref[...] += jnp.sum(nll * m).reshape(1, 1)
    den_ref[...] += jnp.sum(m).reshape(1, 1)


def _head(xf, le, tgt, bt=128):
    return pl.pallas_call(
        _head_kern,
        grid=(T // bt,),
        in_specs=[
            pl.BlockSpec((bt, D), lambda t: (t, 0)),
            pl.BlockSpec((V, D), lambda t: (0, 0)),
            pl.BlockSpec((bt, 1), lambda t: (t, 0)),
        ],
        out_specs=[
            pl.BlockSpec((bt, V), lambda t: (t, 0)),
            pl.BlockSpec((1, 1), lambda t: (0, 0)),
            pl.BlockSpec((1, 1), lambda t: (0, 0)),
        ],
        out_shape=[
            jax.ShapeDtypeStruct((T, V), jnp.float32),
            jax.ShapeDtypeStruct((1, 1), jnp.float32),
            jax.ShapeDtypeStruct((1, 1), jnp.float32),
        ],
    )(xf, le, tgt)


def _embed(params, his, ts, vocab):
    loc = vocab[his]
    poi = loc[..., :28]
    lonlat = loc[..., 28:30]
    rank = loc[..., 30].astype(jnp.int32)
    poi_e = poi @ params['poi_W'].T + params['poi_b']
    ll_e = lonlat @ params['lonlat_W'].T + params['lonlat_b']
    r_e = params['rank_emb'][rank]
    tok = jnp.concatenate([ll_e, r_e, poi_e], axis=-1)
    x = tok + params['time_emb'][ts] + params['wpe'][None, :ts.shape[1], :]
    return x


def kernel(params, his, ts, targets, vocab):
    p = params
    x = _embed(p, his, ts, vocab)[0]
    for li, bp in enumerate(p['blocks']):
        qkv = _ln_mm(x, bp['ln1_g'], bp['ln1_b'], bp['in_proj_W'],
                     bp['in_proj_b'])
        qkvh = qkv.reshape(T, 3 * H, DH).transpose(1, 0, 2)
        a = _attention(qkvh, his)
        a = a.transpose(1, 0, 2).reshape(T, D)
        x = _proj_res(a, bp['out_W'], bp['out_b'], x)
        nz = jax.random.normal(jax.random.fold_in(jax.random.key(42), li),
                               (1, T, E), jnp.float32)[0]
        y, gate = _router(x, bp['ln2_g'], bp['ln2_b'], bp['route_W'],
                          bp['route_b'], bp['noise_W'], bp['noise_b'], nz)
        x = _moe(y, gate, bp['W1'], bp['b1'], bp['W2'], bp['b2'], x)
    xf = _ln_mm(x, p['lnf_g'], p['lnf_b'], p['lm_head_W'],
                jnp.zeros((D,), jnp.float32))
    le = _tower(vocab, p['tower_W'], p['tower_b'])
    logits, num, den = _head(xf, le, targets.reshape(T, 1))
    loss = (num / jnp.maximum(den, 1.0)).reshape(())
    return logits.reshape(1, T, V), loss
